# speedup vs baseline: 5.3457x; 5.3457x over previous
import jax
import jax.numpy as jnp
from jax import lax
from jax.experimental import pallas as pl
from jax.experimental.pallas import tpu as pltpu

_NUM_CLASSES = 10
_K = _NUM_CLASSES - 1
_ROWS_PER_GROUP = 8
_LANES = _K * _ROWS_PER_GROUP
_EPS = 1e-9


def _body(logits_ref, labels_ref, out_ref, acc_ref):
    b = pl.program_id(0)
    nb = pl.num_programs(0)

    @pl.when(b == 0)
    def _init():
        acc_ref[...] = jnp.zeros_like(acc_ref)

    x = logits_ref[...]
    lab = labels_ref[...]

    bb = x.shape[0]
    lane = lax.broadcasted_iota(jnp.int32, (bb, _LANES), 1)
    jpat = lane % _K
    rpat = lane // _K

    s = jax.nn.sigmoid(x)
    s_shift = jnp.concatenate([s[:, 1:], s[:, :1]], axis=1)
    p = s - jnp.where(jpat == _K - 1, jnp.float32(1.0), s_shift)

    logp = jnp.log(p + _EPS)
    log1mp = jnp.log(1.0 - p + _EPS)

    labexp = jnp.zeros((bb, _LANES), jnp.int32)
    for r in range(_ROWS_PER_GROUP):
        labexp = jnp.where(rpat == r, lab[:, r:r + 1], labexp)

    ohf = (jpat == labexp).astype(jnp.float32)
    pe = ohf * logp + (1.0 - ohf) * log1mp

    for c in range(_NUM_CLASSES):
        m = labexp == c
        acc_ref[c:c + 1, :_LANES] += jnp.sum(
            jnp.where(m, pe, 0.0), axis=0, keepdims=True)
        acc_ref[_NUM_CLASSES + c:_NUM_CLASSES + c + 1, :_LANES] += jnp.sum(
            m.astype(jnp.float32), axis=0, keepdims=True)

    @pl.when(b == nb - 1)
    def _finalize():
        acc = acc_ref[...]
        rows = jnp.sum(acc[:, :_LANES], axis=1, keepdims=True)
        ridx = lax.broadcasted_iota(jnp.int32, rows.shape, 0)
        s_c = jnp.where(ridx < _NUM_CLASSES, rows, 0.0)
        cnt_al = jnp.concatenate(
            [rows[_NUM_CLASSES:2 * _NUM_CLASSES] / jnp.float32(_K),
             jnp.zeros((rows.shape[0] - _NUM_CLASSES, 1), jnp.float32)],
            axis=0)
        total = jnp.sum(cnt_al)
        valid = ridx < _NUM_CLASSES
        w = cnt_al / total
        w = jnp.where(w == 0.0, jnp.float32(1.0), w)
        inv = jnp.where(valid, 1.0 / w, 0.0)
        z = jnp.sum(inv)
        loss = -jnp.sum(inv * s_c) / (z * total)
        out_ref[...] = jnp.full_like(out_ref, loss)


def kernel(logits, labels):
    n = logits.shape[0]
    groups = n // _ROWS_PER_GROUP
    lg = logits.reshape(groups, _LANES)
    lb = labels.astype(jnp.int32).reshape(groups, _ROWS_PER_GROUP)

    bb = 1024
    grid = (groups // bb,)
    out = pl.pallas_call(
        _body,
        grid=grid,
        in_specs=[
            pl.BlockSpec((bb, _LANES), lambda i: (i, 0)),
            pl.BlockSpec((bb, _ROWS_PER_GROUP), lambda i: (i, 0)),
        ],
        out_specs=pl.BlockSpec((8, 128), lambda i: (0, 0)),
        out_shape=jax.ShapeDtypeStruct((8, 128), jnp.float32),
        scratch_shapes=[pltpu.VMEM((32, 128), jnp.float32)],
        compiler_params=pltpu.CompilerParams(
            dimension_semantics=("arbitrary",)),
    )(lg, lb)
    return out[0, 0]


# speedup vs baseline: 7.4830x; 1.3998x over previous
import jax
import jax.numpy as jnp
from jax import lax
from jax.experimental import pallas as pl
from jax.experimental.pallas import tpu as pltpu

_NUM_CLASSES = 10
_K = _NUM_CLASSES - 1
_ROWS_PER_GROUP = 8
_LANES = _K * _ROWS_PER_GROUP
_EPS = 1e-9


def _counts_body(lab_ref, invw_ref):
    lab = lab_ref[...]
    lane = lax.broadcasted_iota(jnp.int32, (1, 128), 1)
    cnts = jnp.zeros((1, 128), jnp.float32)
    total = jnp.float32(0.0)
    for c in range(_NUM_CLASSES):
        sc = jnp.sum((lab == c).astype(jnp.float32))
        cnts = jnp.where(lane == c, sc, cnts)
        total = total + sc
    valid = lane < _NUM_CLASSES
    w = cnts / total
    w = jnp.where(valid & (w == 0.0), jnp.float32(1.0), w)
    inv = jnp.where(valid, 1.0 / w, 0.0)
    invn = inv / jnp.sum(inv)
    invw_ref[...] = jnp.broadcast_to(invn, invw_ref.shape)


def _dense_body(logits_ref, labels_ref, invw_ref, out_ref, acc_ref):
    b = pl.program_id(0)
    nb = pl.num_programs(0)

    @pl.when(b == 0)
    def _init():
        acc_ref[...] = jnp.zeros_like(acc_ref)

    x = logits_ref[...]
    lab = labels_ref[...]

    bb = x.shape[0]
    lane = lax.broadcasted_iota(jnp.int32, (bb, _LANES), 1)
    jpat = lane % _K
    rpat = lane // _K

    labexp = jnp.take_along_axis(lab, rpat, axis=1)

    s = jax.nn.sigmoid(x)
    s_shift = jnp.concatenate([s[:, 1:], s[:, :1]], axis=1)
    p = s - jnp.where(jpat == _K - 1, jnp.float32(1.0), s_shift)

    logp = jnp.log(p + _EPS)
    log1mp = jnp.log(1.0 - p + _EPS)

    ohf = (jpat == labexp).astype(jnp.float32)
    pe = ohf * logp + (1.0 - ohf) * log1mp

    invw_b = jnp.broadcast_to(invw_ref[0:1, :_LANES], (bb, _LANES))
    wexp = jnp.take_along_axis(invw_b, labexp, axis=1)

    acc_ref[0:1, :_LANES] += jnp.sum(wexp * pe, axis=0, keepdims=True)

    @pl.when(b == nb - 1)
    def _finalize():
        n_rows = jnp.float32(nb) * bb * _ROWS_PER_GROUP
        loss = -jnp.sum(acc_ref[0:1, :_LANES]) / n_rows
        out_ref[...] = jnp.full_like(out_ref, loss)


def kernel(logits, labels):
    n = logits.shape[0]
    groups = n // _ROWS_PER_GROUP
    lg = logits.reshape(groups, _LANES)
    lb32 = labels.astype(jnp.int32)
    lb = lb32.reshape(groups, _ROWS_PER_GROUP)
    lab_dense = lb32.reshape(n // 128, 128)

    invw = pl.pallas_call(
        _counts_body,
        out_specs=pl.BlockSpec((8, 128), lambda: (0, 0)),
        out_shape=jax.ShapeDtypeStruct((8, 128), jnp.float32),
    )(lab_dense)

    bb = 1024
    grid = (groups // bb,)
    out = pl.pallas_call(
        _dense_body,
        grid=grid,
        in_specs=[
            pl.BlockSpec((bb, _LANES), lambda i: (i, 0)),
            pl.BlockSpec((bb, _ROWS_PER_GROUP), lambda i: (i, 0)),
            pl.BlockSpec((8, 128), lambda i: (0, 0)),
        ],
        out_specs=pl.BlockSpec((8, 128), lambda i: (0, 0)),
        out_shape=jax.ShapeDtypeStruct((8, 128), jnp.float32),
        scratch_shapes=[pltpu.VMEM((8, 128), jnp.float32)],
        compiler_params=pltpu.CompilerParams(
            dimension_semantics=("arbitrary",)),
    )(lg, lb, invw)
    return out[0, 0]


# speedup vs baseline: 8.5590x; 1.1438x over previous
import jax
import jax.numpy as jnp
from jax import lax
from jax.experimental import pallas as pl
from jax.experimental.pallas import tpu as pltpu

_NUM_CLASSES = 10
_K = _NUM_CLASSES - 1
_ROWS_PER_SUBLANE = 128
_W = _K * _ROWS_PER_SUBLANE
_EPS = 1e-9


def _counts_body(lab_ref, invw_ref):
    lab = lab_ref[...]
    lane = lax.broadcasted_iota(jnp.int32, (1, 128), 1)
    cnts = jnp.zeros((1, 128), jnp.float32)
    total = jnp.float32(0.0)
    for c in range(_NUM_CLASSES):
        sc = jnp.sum((lab == c).astype(jnp.float32))
        cnts = jnp.where(lane == c, sc, cnts)
        total = total + sc
    valid = lane < _NUM_CLASSES
    w = cnts / total
    w = jnp.where(valid & (w == 0.0), jnp.float32(1.0), w)
    inv = jnp.where(valid, 1.0 / w, 0.0)
    invn = inv / jnp.sum(inv)
    invw_ref[...] = jnp.broadcast_to(invn, invw_ref.shape)


def _chunk_gather(src, idx):
    outs = [
        jnp.take_along_axis(src, idx[:, 128 * v:128 * (v + 1)], axis=1)
        for v in range(_K)
    ]
    return jnp.concatenate(outs, axis=1)


def _dense_body(logits_ref, labels_ref, invw_ref, out_ref, acc_ref):
    b = pl.program_id(0)
    nb = pl.num_programs(0)

    @pl.when(b == 0)
    def _init():
        acc_ref[...] = jnp.zeros_like(acc_ref)

    x = logits_ref[...]
    lab = labels_ref[...]

    bb = x.shape[0]
    lane = lax.broadcasted_iota(jnp.int32, (bb, _W), 1)
    jpat = lane % _K
    rpat = lane // _K

    labexp = _chunk_gather(lab, rpat)

    s = jax.nn.sigmoid(x)
    s_shift = jnp.concatenate([s[:, 1:], s[:, :1]], axis=1)
    p = s - jnp.where(jpat == _K - 1, jnp.float32(1.0), s_shift)

    logp = jnp.log(p + _EPS)
    log1mp = jnp.log(1.0 - p + _EPS)

    ohf = (jpat == labexp).astype(jnp.float32)
    pe = ohf * logp + (1.0 - ohf) * log1mp

    invw_b = jnp.broadcast_to(invw_ref[0:1, :], (bb, 128))
    wexp = _chunk_gather(invw_b, labexp)

    acc_ref[0:1, :] += jnp.sum(wexp * pe, axis=0, keepdims=True)

    @pl.when(b == nb - 1)
    def _finalize():
        n_rows = jnp.float32(nb) * bb * _ROWS_PER_SUBLANE
        loss = -jnp.sum(acc_ref[0:1, :]) / n_rows
        out_ref[...] = jnp.full_like(out_ref, loss)


def kernel(logits, labels):
    n = logits.shape[0]
    sl = n // _ROWS_PER_SUBLANE
    lg = logits.reshape(sl, _W)
    lab_dense = labels.astype(jnp.int32).reshape(sl, 128)

    invw = pl.pallas_call(
        _counts_body,
        out_specs=pl.BlockSpec((8, 128), lambda: (0, 0)),
        out_shape=jax.ShapeDtypeStruct((8, 128), jnp.float32),
    )(lab_dense)

    bb = 128
    grid = (sl // bb,)
    out = pl.pallas_call(
        _dense_body,
        grid=grid,
        in_specs=[
            pl.BlockSpec((bb, _W), lambda i: (i, 0)),
            pl.BlockSpec((bb, 128), lambda i: (i, 0)),
            pl.BlockSpec((8, 128), lambda i: (0, 0)),
        ],
        out_specs=pl.BlockSpec((8, 128), lambda i: (0, 0)),
        out_shape=jax.ShapeDtypeStruct((8, 128), jnp.float32),
        scratch_shapes=[pltpu.VMEM((8, _W), jnp.float32)],
        compiler_params=pltpu.CompilerParams(
            dimension_semantics=("arbitrary",)),
    )(lg, lab_dense, invw)
    return out[0, 0]
